# baseline (speedup 1.0000x reference)
import jax
import jax.numpy as jnp
from jax.experimental import pallas as pl
from jax.experimental.pallas import tpu as pltpu

_D_EMB = 64


def _pos_block_kernel(e_ref, w_ref, b_ref, o_ref, acc_ref):
    j = pl.program_id(1)

    @pl.when(j == 0)
    def _():
        acc_ref[...] = (
            jnp.dot(e_ref[...], w_ref[...], preferred_element_type=jnp.float32)
            + b_ref[...]
        )

    o_ref[0] = acc_ref[...]


def kernel(x, embedding, W, b):
    B, S, D = x.shape
    bs = 2048
    ns = S // bs
    b2 = b.reshape(1, D)
    return pl.pallas_call(
        _pos_block_kernel,
        grid=(ns, B),
        in_specs=[
            pl.BlockSpec((bs, _D_EMB), lambda i, j: (i, 0)),
            pl.BlockSpec((_D_EMB, D), lambda i, j: (0, 0)),
            pl.BlockSpec((1, D), lambda i, j: (0, 0)),
        ],
        out_specs=pl.BlockSpec((1, bs, D), lambda i, j: (j, i, 0)),
        out_shape=jax.ShapeDtypeStruct((B, S, D), jnp.float32),
        scratch_shapes=[pltpu.VMEM((bs, D), jnp.float32)],
        compiler_params=pltpu.CompilerParams(
            dimension_semantics=("arbitrary", "arbitrary"),
        ),
    )(embedding, W, b2)


# speedup vs baseline: 1.0078x; 1.0078x over previous
import jax
import jax.numpy as jnp
from jax.experimental import pallas as pl
from jax.experimental.pallas import tpu as pltpu

_D_EMB = 64
_BS = 2048


def _copies(acc_ref, o_ref, sem_ref, step, bs, batch):
    buf = step % 2
    return [
        pltpu.make_async_copy(
            acc_ref.at[buf],
            o_ref.at[j, pl.ds(step * bs, bs), :],
            sem_ref.at[buf, j],
        )
        for j in range(batch)
    ]


def _pos_block_kernel(e_ref, w_ref, b_ref, o_ref, acc_ref, sem_ref):
    i = pl.program_id(0)
    ns = pl.num_programs(0)
    batch = o_ref.shape[0]
    bs = e_ref.shape[0]
    p = i % 2

    @pl.when(i >= 2)
    def _():
        for c in _copies(acc_ref, o_ref, sem_ref, i - 2, bs, batch):
            c.wait()

    acc_ref[p] = (
        jnp.dot(e_ref[...], w_ref[...], preferred_element_type=jnp.float32)
        + b_ref[...]
    )
    for c in _copies(acc_ref, o_ref, sem_ref, i, bs, batch):
        c.start()

    @pl.when(i == ns - 1)
    def _():
        for step in (i - 1, i):
            @pl.when(step >= 0)
            def _():
                for c in _copies(acc_ref, o_ref, sem_ref, step, bs, batch):
                    c.wait()


def kernel(x, embedding, W, b):
    B, S, D = x.shape
    bs = _BS
    ns = S // bs
    b2 = b.reshape(1, D)
    return pl.pallas_call(
        _pos_block_kernel,
        grid=(ns,),
        in_specs=[
            pl.BlockSpec((bs, _D_EMB), lambda i: (i, 0)),
            pl.BlockSpec((_D_EMB, D), lambda i: (0, 0)),
            pl.BlockSpec((1, D), lambda i: (0, 0)),
        ],
        out_specs=pl.BlockSpec(memory_space=pltpu.MemorySpace.HBM),
        out_shape=jax.ShapeDtypeStruct((B, S, D), jnp.float32),
        scratch_shapes=[
            pltpu.VMEM((2, bs, D), jnp.float32),
            pltpu.SemaphoreType.DMA((2, B)),
        ],
        compiler_params=pltpu.CompilerParams(
            dimension_semantics=("arbitrary",),
        ),
    )(embedding, W, b2)


# speedup vs baseline: 1.0591x; 1.0509x over previous
import jax
import jax.numpy as jnp
from jax.experimental import pallas as pl
from jax.experimental.pallas import tpu as pltpu

_D_EMB = 64
_BS = 512
_NBUF = 4


def _copies(acc_ref, o_ref, sem_ref, step, bs, batch):
    buf = step % _NBUF
    return [
        pltpu.make_async_copy(
            acc_ref.at[buf],
            o_ref.at[j, pl.ds(step * bs, bs), :],
            sem_ref.at[buf, j],
        )
        for j in range(batch)
    ]


def _pos_block_kernel(e_ref, w_ref, b_ref, o_ref, acc_ref, sem_ref):
    i = pl.program_id(0)
    ns = pl.num_programs(0)
    batch = o_ref.shape[0]
    bs = e_ref.shape[0]
    p = i % _NBUF

    @pl.when(i >= _NBUF)
    def _():
        for c in _copies(acc_ref, o_ref, sem_ref, i - _NBUF, bs, batch):
            c.wait()

    acc_ref[p] = (
        jnp.dot(e_ref[...], w_ref[...], preferred_element_type=jnp.float32)
        + b_ref[...]
    )
    for c in _copies(acc_ref, o_ref, sem_ref, i, bs, batch):
        c.start()

    @pl.when(i == ns - 1)
    def _():
        for back in range(_NBUF - 1, -1, -1):
            step = i - back

            @pl.when(step >= 0)
            def _():
                for c in _copies(acc_ref, o_ref, sem_ref, step, bs, batch):
                    c.wait()


def kernel(x, embedding, W, b):
    B, S, D = x.shape
    bs = _BS
    ns = S // bs
    b2 = b.reshape(1, D)
    return pl.pallas_call(
        _pos_block_kernel,
        grid=(ns,),
        in_specs=[
            pl.BlockSpec((bs, _D_EMB), lambda i: (i, 0)),
            pl.BlockSpec((_D_EMB, D), lambda i: (0, 0)),
            pl.BlockSpec((1, D), lambda i: (0, 0)),
        ],
        out_specs=pl.BlockSpec(memory_space=pltpu.MemorySpace.HBM),
        out_shape=jax.ShapeDtypeStruct((B, S, D), jnp.float32),
        scratch_shapes=[
            pltpu.VMEM((_NBUF, bs, D), jnp.float32),
            pltpu.SemaphoreType.DMA((_NBUF, B)),
        ],
        compiler_params=pltpu.CompilerParams(
            dimension_semantics=("arbitrary",),
        ),
    )(embedding, W, b2)
